# baseline (speedup 1.0000x reference)
import functools

import jax
import jax.numpy as jnp
from jax import lax
from jax.experimental import pallas as pl
from jax.experimental.pallas import tpu as pltpu, tpu_sc as plsc

B = 16384
MEM = 172
EF = 172
MP = 176
TENC = 100
N_NODES = 100000
N_EDGES = 400000

_NC = 2
_NS = 16
_NW = _NC * _NS

_RCH = 64
_NCHUNK = B // _RCH
_TOTAL = 3 * _NCHUNK
_GT = _NW - 2
_PER_TILE = -(-_TOTAL // _GT)

_SG = 512
_NSG = B // _SG

_CP = pltpu.CompilerParams(use_tc_tiling_on_sc=False,
                           needs_layout_passes=False)
_MESH = plsc.VectorSubcoreMesh(core_axis_name="c", subcore_axis_name="s")


def _lanes():
    return lax.iota(jnp.int32, 16)


_TBLK = 2048


def _tc_transpose_bf16(x_t):
    d, n = x_t.shape
    grid = -(-n // _TBLK)
    eye = jnp.eye(d, MP, dtype=x_t.dtype)

    def body(x_ref, eye_ref, o_ref):
        o_ref[...] = lax.dot_general(
            x_ref[...], eye_ref[...], (((0,), (0,)), ((), ())),
            preferred_element_type=jnp.float32).astype(jnp.bfloat16)

    return pl.pallas_call(
        body,
        grid=(grid,),
        in_specs=[pl.BlockSpec((d, _TBLK), lambda i: (0, i)),
                  pl.BlockSpec((d, MP), lambda i: (0, 0))],
        out_specs=pl.BlockSpec((_TBLK, MP), lambda i: (i, 0)),
        out_shape=jax.ShapeDtypeStruct((n, MP), jnp.bfloat16),
    )(x_t, eye)


def _tc_transpose(x_t):
    d, n = x_t.shape
    grid = -(-n // _TBLK)
    eye = jnp.eye(d, MP, dtype=x_t.dtype)

    def body(x_ref, eye_ref, o_ref):
        o_ref[...] = lax.dot_general(
            x_ref[...], eye_ref[...], (((0,), (0,)), ((), ())),
            preferred_element_type=jnp.float32)

    return pl.pallas_call(
        body,
        grid=(grid,),
        in_specs=[pl.BlockSpec((d, _TBLK), lambda i: (0, i)),
                  pl.BlockSpec((d, MP), lambda i: (0, 0))],
        out_specs=pl.BlockSpec((_TBLK, MP), lambda i: (i, 0)),
        out_shape=jax.ShapeDtypeStruct((n, MP), x_t.dtype),
    )(x_t, eye)


def _tc_transpose_back(x):
    n, d = x.shape
    grid = -(-n // _TBLK)
    eye = jnp.eye(MEM, MP, dtype=x.dtype)

    def body(x_ref, eye_ref, o_ref):
        o_ref[...] = lax.dot_general(
            eye_ref[...], x_ref[...], (((1,), (1,)), ((), ())),
            preferred_element_type=jnp.float32)

    return pl.pallas_call(
        body,
        grid=(grid,),
        in_specs=[pl.BlockSpec((_TBLK, MP), lambda i: (i, 0)),
                  pl.BlockSpec((MEM, MP), lambda i: (0, 0))],
        out_specs=pl.BlockSpec((MEM, _TBLK), lambda i: (0, i)),
        out_shape=jax.ShapeDtypeStruct((MEM, n), x.dtype),
    )(x, eye)


def _sc_gather(sources, destinations, edge_idxs, node_mem, edge_features,
               lu_i32):
    @functools.partial(
        pl.kernel,
        mesh=_MESH,
        compiler_params=_CP,
        out_type=(
            jax.ShapeDtypeStruct((B, MP), jnp.float32),
            jax.ShapeDtypeStruct((B, MP), jnp.float32),
            jax.ShapeDtypeStruct((B, MP), jnp.bfloat16),
            jax.ShapeDtypeStruct((B,), jnp.int32),
            jax.ShapeDtypeStruct((B,), jnp.int32),
        ),
        scratch_types=[
            pltpu.VMEM((N_NODES,), jnp.int32),
            pltpu.VMEM((_RCH, MP), jnp.float32),
            pltpu.VMEM((_RCH, MP), jnp.bfloat16),
            pltpu.VMEM((_RCH,), jnp.int32),
            pltpu.VMEM((_SG,), jnp.int32),
            pltpu.VMEM((_SG,), jnp.int32),
            pltpu.SemaphoreType.DMA,
        ],
    )
    def k(src_hbm, dst_hbm, eidx_hbm, mem_f, ef_f, lu_hbm,
          srcg_hbm, dstg_hbm, efg_hbm, tlu_hbm, scat_hbm,
          table, rowbuf, rowbuf2, idxv, sbuf, outb, sem):
        wid = lax.axis_index("s") * _NC + lax.axis_index("c")
        lanes = _lanes()

        @pl.when(wid >= 2)
        def _():
            for cch in range(_PER_TILE):
                q = (wid - 2) * _PER_TILE + cch

                @pl.when(q < _TOTAL)
                def _():
                    typ = q // _NCHUNK
                    off = (q % _NCHUNK) * _RCH

                    @pl.when(typ == 0)
                    def _():
                        pltpu.sync_copy(src_hbm.at[pl.ds(off, _RCH)], idxv)

                    @pl.when(typ == 1)
                    def _():
                        pltpu.sync_copy(dst_hbm.at[pl.ds(off, _RCH)], idxv)

                    @pl.when(typ == 2)
                    def _():
                        pltpu.sync_copy(eidx_hbm.at[pl.ds(off, _RCH)], idxv)

                    def fire(j, carry):
                        g = (j // 16) * 16
                        v = idxv[pl.ds(g, 16)]
                        r = jnp.max(jnp.where(lanes == j - g, v,
                                              jnp.int32(-1)))

                        @pl.when(typ <= 1)
                        def _():
                            pltpu.async_copy(mem_f.at[pl.ds(r, 1)],
                                             rowbuf.at[pl.ds(j, 1)], sem)

                        @pl.when(typ == 2)
                        def _():
                            pltpu.async_copy(ef_f.at[pl.ds(r, 1)],
                                             rowbuf2.at[pl.ds(j, 1)], sem)

                        return carry

                    lax.fori_loop(0, _RCH, fire, 0)
                    @pl.when(typ <= 1)
                    def _():
                        pltpu.make_async_copy(mem_f.at[pl.ds(0, _RCH)],
                                              rowbuf, sem).wait()

                    @pl.when(typ == 2)
                    def _():
                        pltpu.make_async_copy(ef_f.at[pl.ds(0, _RCH)],
                                              rowbuf2, sem).wait()

                    @pl.when(typ == 0)
                    def _():
                        pltpu.sync_copy(rowbuf, srcg_hbm.at[pl.ds(off, _RCH)])

                    @pl.when(typ == 1)
                    def _():
                        pltpu.sync_copy(rowbuf, dstg_hbm.at[pl.ds(off, _RCH)])

                    @pl.when(typ == 2)
                    def _():
                        pltpu.sync_copy(rowbuf2, efg_hbm.at[pl.ds(off, _RCH)])

        @pl.when(wid == 1)
        def _():
            pltpu.sync_copy(lu_hbm, table)
            for g in range(_NSG):
                pltpu.sync_copy(src_hbm.at[pl.ds(g * _SG, _SG)], sbuf)

                def body(kk, carry):
                    s = sbuf[pl.ds(kk * 16, 16)]
                    outb[pl.ds(kk * 16, 16)] = plsc.load_gather(table, [s])
                    return carry

                lax.fori_loop(0, _SG // 16, body, 0)
                pltpu.sync_copy(outb, tlu_hbm.at[pl.ds(g * _SG, _SG)])

        @pl.when(wid == 0)
        def _():
            for g in range(_NSG):
                pltpu.sync_copy(src_hbm.at[pl.ds(g * _SG, _SG)], sbuf)

                def p1(kk, carry):
                    s = sbuf[pl.ds(kk * 16, 16)]
                    plsc.store_scatter(table, [s],
                                       jnp.full((16,), -1, jnp.int32))
                    return carry

                lax.fori_loop(0, _SG // 16, p1, 0)

            for g in range(_NSG):
                pltpu.sync_copy(src_hbm.at[pl.ds(g * _SG, _SG)], sbuf)

                def p2(kk, carry):
                    s = sbuf[pl.ds(kk * 16, 16)]
                    pos = g * _SG + kk * 16 + lanes
                    key = s * B + pos
                    sk, sv = plsc.sort_key_val(key, s, descending=True)
                    prev = lax.gather(
                        sv, jnp.maximum(lanes - 1, 0)[:, None],
                        dimension_numbers=lax.GatherDimensionNumbers(
                            offset_dims=(), collapsed_slice_dims=(0,),
                            start_index_map=(0,)),
                        slice_sizes=(1,),
                        mode=lax.GatherScatterMode.PROMISE_IN_BOUNDS)
                    firstm = (lanes == 0) | (sv != prev)
                    cur = plsc.load_gather(table, [sv])
                    plsc.store_scatter(table, [sv], sk,
                                       mask=firstm & (sk > cur))
                    return carry

                lax.fori_loop(0, _SG // 16, p2, 0)

            for g in range(_NSG):
                pltpu.sync_copy(src_hbm.at[pl.ds(g * _SG, _SG)], sbuf)

                def p3(kk, carry):
                    s = sbuf[pl.ds(kk * 16, 16)]
                    pos = g * _SG + kk * 16 + lanes
                    key = s * B + pos
                    cur = plsc.load_gather(table, [s])
                    outb[pl.ds(kk * 16, 16)] = jnp.where(cur == key, s,
                                                         jnp.int32(-1))
                    return carry

                lax.fori_loop(0, _SG // 16, p3, 0)
                pltpu.sync_copy(outb, scat_hbm.at[pl.ds(g * _SG, _SG)])

    return k(sources, destinations, edge_idxs, node_mem, edge_features,
             lu_i32)


_BLK = 1024
_GRID = B // _BLK


def _tc_dense(src_g, dst_g, ef_g, tlu, ts, w_src, w_dst, w_ef, w_t, bias,
              w_time_row, b_time_row):
    def body(src_ref, dst_ref, ef_ref, tlu_ref, ts_ref, wsrc_ref, wdst_ref,
             wef_ref, wt_ref, bias_ref, wtime_ref, btime_ref, out_ref):
        td = ts_ref[0, 0, :] - tlu_ref[0, 0, :]
        tenc = jnp.cos(td[:, None] * wtime_ref[0, :][None, :]
                       + btime_ref[0, :][None, :])
        h = jnp.dot(src_ref[...], wsrc_ref[...],
                    preferred_element_type=jnp.float32)
        h += jnp.dot(dst_ref[...], wdst_ref[...],
                     preferred_element_type=jnp.float32)
        h += jnp.dot(ef_ref[...], wef_ref[...],
                     preferred_element_type=jnp.float32)
        h += jnp.dot(tenc, wt_ref[...], preferred_element_type=jnp.float32)
        out_ref[...] = jnp.tanh(h + bias_ref[0, :][None, :])

    full = lambda shape: pl.BlockSpec(shape, lambda i: (0, 0))
    row = lambda shape: pl.BlockSpec(shape, lambda i: (i, 0))
    return pl.pallas_call(
        body,
        grid=(_GRID,),
        in_specs=[
            row((_BLK, MP)), row((_BLK, MP)), row((_BLK, MP)),
            pl.BlockSpec((1, 1, _BLK), lambda i: (i, 0, 0)),
            pl.BlockSpec((1, 1, _BLK), lambda i: (i, 0, 0)),
            full((MP, MP)), full((MP, MP)), full((MP, MP)),
            full((TENC, MP)), full((1, MP)), full((1, TENC)),
            full((1, TENC)),
        ],
        out_specs=row((_BLK, MP)),
        out_shape=jax.ShapeDtypeStruct((B, MP), jnp.float32),
    )(src_g, dst_g, ef_g, tlu.reshape(_GRID, 1, _BLK),
      ts.reshape(_GRID, 1, _BLK),
      w_src, w_dst, w_ef, w_t, bias, w_time_row, b_time_row)


_RANGE = N_NODES // _NW
_RBLK = 128
_NBLK = -(-_RANGE // _RBLK)
_WPAD = _NBLK * _RBLK


def _sc_scatter(h_new, scat_idx, node_mem):
    @functools.partial(
        pl.kernel,
        mesh=_MESH,
        compiler_params=_CP,
        out_type=jax.ShapeDtypeStruct((N_NODES, MP), jnp.float32),
        scratch_types=[
            pltpu.VMEM((_RBLK, MP), jnp.float32),
            pltpu.VMEM((_RBLK, MP), jnp.float32),
            pltpu.VMEM((_WPAD,), jnp.int32),
            pltpu.VMEM((_SG,), jnp.int32),
            pltpu.VMEM((1, MP), jnp.float32),
            pltpu.SemaphoreType.DMA,
            pltpu.SemaphoreType.DMA,
            pltpu.SemaphoreType.DMA,
        ],
    )
    def k(h_f, scat_hbm, mem_f, out_f, blockbuf_a, blockbuf_b, winmap,
          sbuf, dummy, sem, sem2a, sem2b):
        wid = lax.axis_index("s") * _NC + lax.axis_index("c")
        lanes = _lanes()
        base = wid * _RANGE

        def wm0(kk, carry):
            winmap[pl.ds(kk * 16, 16)] = jnp.full((16,), -1, jnp.int32)
            return carry

        lax.fori_loop(0, _WPAD // 16, wm0, 0)

        for g in range(_NSG):
            pltpu.sync_copy(scat_hbm.at[pl.ds(g * _SG, _SG)], sbuf)

            def scan(kk, carry):
                r = sbuf[pl.ds(kk * 16, 16)]
                pos = g * _SG + kk * 16 + lanes
                inr = (r >= base) & (r < base + _RANGE)
                plsc.store_scatter(winmap, [jnp.where(inr, r - base, 0)],
                                   pos, mask=inr)
                return carry

            lax.fori_loop(0, _SG // 16, scan, 0)

        bufs = (blockbuf_a, blockbuf_b)
        sems = (sem2a, sem2b)

        def bsize(b):
            return _RBLK if b < _NBLK - 1 else _RANGE - (_NBLK - 1) * _RBLK

        def load(b):
            pltpu.async_copy(
                mem_f.at[pl.ds(base + b * _RBLK, bsize(b))],
                bufs[b % 2].at[pl.ds(0, bsize(b))], sems[b % 2])

        load(0)
        for b in range(_NBLK):
            size = bsize(b)
            boff = b * _RBLK
            cur = bufs[b % 2]
            pltpu.make_async_copy(mem_f.at[pl.ds(0, size)],
                                  cur.at[pl.ds(0, size)],
                                  sems[b % 2]).wait()
            if b + 1 < _NBLK:
                load(b + 1)

            def fire(j, n):
                g = (j // 16) * 16
                v = winmap[pl.ds(boff + g, 16)]
                i_s = jnp.max(jnp.where(lanes == j - g, v, jnp.int32(-1)))

                @pl.when(i_s >= 0)
                def _():
                    pltpu.async_copy(h_f.at[pl.ds(i_s, 1)],
                                     cur.at[pl.ds(j, 1)], sem)

                return n + jnp.where(i_s >= 0, 1, 0)

            n = lax.fori_loop(0, size, fire, jnp.int32(0))

            def drain(j, carry):
                pltpu.make_async_copy(h_f.at[pl.ds(0, 1)], dummy,
                                      sem).wait()
                return carry

            lax.fori_loop(0, n, drain, 0)
            pltpu.sync_copy(cur.at[pl.ds(0, size)],
                            out_f.at[pl.ds(base + boff, size)])

    return k(h_new, scat_idx, node_mem)


def kernel(sources, destinations, timestamps, edge_idxs, edge_features,
           node_mem, last_updated, w_time, b_time, W_ih, W_hh, b_ih, b_hh):
    lu_i32 = lax.bitcast_convert_type(last_updated, jnp.int32)

    nm_row = _tc_transpose(node_mem.T)
    ef_row = _tc_transpose_bf16(edge_features.T)

    src_g, dst_g, ef_g, tlu_i32, scat_idx = _sc_gather(
        sources, destinations, edge_idxs, nm_row, ef_row, lu_i32)
    tlu = lax.bitcast_convert_type(tlu_i32, jnp.float32)

    pad = lambda w: jnp.pad(w, ((0, MP - w.shape[0]), (0, MP - w.shape[1])))
    w_src = pad((W_ih[:, :MEM] + W_hh).T)
    w_dst = pad(W_ih[:, MEM:2 * MEM].T)
    w_ef = pad(W_ih[:, 2 * MEM:2 * MEM + EF].T).astype(jnp.bfloat16)
    w_t = jnp.pad(W_ih[:, 2 * MEM + EF:].T, ((0, 0), (0, MP - MEM)))
    bias = jnp.pad(b_ih + b_hh, (0, MP - MEM)).reshape(1, MP)
    w_time_row = w_time[:, 0].reshape(1, TENC)
    b_time_row = b_time.reshape(1, TENC)

    h_new = _tc_dense(src_g, dst_g, ef_g, tlu, timestamps, w_src, w_dst,
                      w_ef, w_t, bias, w_time_row, b_time_row)

    out = _sc_scatter(h_new, scat_idx, nm_row)
    return _tc_transpose_back(out).T


# speedup vs baseline: 1.1869x; 1.1869x over previous
import functools

import jax
import jax.numpy as jnp
from jax import lax
from jax.experimental import pallas as pl
from jax.experimental.pallas import tpu as pltpu, tpu_sc as plsc

B = 16384
MEM = 172
EF = 172
MP = 176
TENC = 100
N_NODES = 100000
N_EDGES = 400000

_NC = 2
_NS = 16
_NW = _NC * _NS

_RCH = 128
_NCHUNK = B // _RCH
_TOTAL = 2 * _NCHUNK
_GT = _NW - 2
_PER_TILE = -(-_TOTAL // _GT)

_SG = 512
_NSG = B // _SG

_CP = pltpu.CompilerParams(use_tc_tiling_on_sc=False,
                           needs_layout_passes=False)
_MESH = plsc.VectorSubcoreMesh(core_axis_name="c", subcore_axis_name="s")


def _lanes():
    return lax.iota(jnp.int32, 16)


_TBLK = 2048


def _tc_transpose(x_t):
    d, n = x_t.shape
    grid = -(-n // _TBLK)
    eye = jnp.eye(d, MP, dtype=x_t.dtype)

    def body(x_ref, eye_ref, o_ref):
        o_ref[...] = lax.dot_general(
            x_ref[...], eye_ref[...], (((0,), (0,)), ((), ())),
            preferred_element_type=jnp.float32)

    return pl.pallas_call(
        body,
        grid=(grid,),
        in_specs=[pl.BlockSpec((d, _TBLK), lambda i: (0, i)),
                  pl.BlockSpec((d, MP), lambda i: (0, 0))],
        out_specs=pl.BlockSpec((_TBLK, MP), lambda i: (i, 0)),
        out_shape=jax.ShapeDtypeStruct((n, MP), x_t.dtype),
    )(x_t, eye)


def _tc_transpose_back(x):
    n, d = x.shape
    grid = -(-n // _TBLK)
    eye = jnp.eye(MEM, MP, dtype=x.dtype)

    def body(x_ref, eye_ref, o_ref):
        o_ref[...] = lax.dot_general(
            eye_ref[...], x_ref[...], (((1,), (1,)), ((), ())),
            preferred_element_type=jnp.float32)

    return pl.pallas_call(
        body,
        grid=(grid,),
        in_specs=[pl.BlockSpec((_TBLK, MP), lambda i: (i, 0)),
                  pl.BlockSpec((MEM, MP), lambda i: (0, 0))],
        out_specs=pl.BlockSpec((MEM, _TBLK), lambda i: (0, i)),
        out_shape=jax.ShapeDtypeStruct((MEM, n), x.dtype),
    )(x, eye)


def _sc_gather(sources, destinations, node_mem, lu_i32):
    @functools.partial(
        pl.kernel,
        mesh=_MESH,
        compiler_params=_CP,
        out_type=(
            jax.ShapeDtypeStruct((B, MP), jnp.float32),
            jax.ShapeDtypeStruct((B, MP), jnp.float32),
            jax.ShapeDtypeStruct((B,), jnp.int32),
            jax.ShapeDtypeStruct((B,), jnp.int32),
        ),
        scratch_types=[
            pltpu.VMEM((N_NODES,), jnp.int32),
            pltpu.VMEM((_RCH, MP), jnp.float32),
            pltpu.VMEM((_RCH,), jnp.int32),
            pltpu.VMEM((_SG,), jnp.int32),
            pltpu.VMEM((_SG,), jnp.int32),
            pltpu.SemaphoreType.DMA,
        ],
    )
    def k(src_hbm, dst_hbm, mem_f, lu_hbm,
          srcg_hbm, dstg_hbm, tlu_hbm, scat_hbm,
          table, rowbuf, idxv, sbuf, outb, sem):
        wid = lax.axis_index("s") * _NC + lax.axis_index("c")
        lanes = _lanes()

        @pl.when(wid >= 2)
        def _():
            for cch in range(_PER_TILE):
                q = (wid - 2) * _PER_TILE + cch

                @pl.when(q < _TOTAL)
                def _():
                    typ = q // _NCHUNK
                    off = (q % _NCHUNK) * _RCH

                    @pl.when(typ == 0)
                    def _():
                        pltpu.sync_copy(src_hbm.at[pl.ds(off, _RCH)], idxv)

                    @pl.when(typ == 1)
                    def _():
                        pltpu.sync_copy(dst_hbm.at[pl.ds(off, _RCH)], idxv)

                    def fire(j, carry):
                        g = (j // 16) * 16
                        v = idxv[pl.ds(g, 16)]
                        r = jnp.max(jnp.where(lanes == j - g, v,
                                              jnp.int32(-1)))
                        pltpu.async_copy(mem_f.at[pl.ds(r, 1)],
                                         rowbuf.at[pl.ds(j, 1)], sem)
                        return carry

                    lax.fori_loop(0, _RCH, fire, 0)
                    pltpu.make_async_copy(mem_f.at[pl.ds(0, _RCH)],
                                          rowbuf, sem).wait()

                    @pl.when(typ == 0)
                    def _():
                        pltpu.sync_copy(rowbuf, srcg_hbm.at[pl.ds(off, _RCH)])

                    @pl.when(typ == 1)
                    def _():
                        pltpu.sync_copy(rowbuf, dstg_hbm.at[pl.ds(off, _RCH)])

        @pl.when(wid == 1)
        def _():
            pltpu.sync_copy(lu_hbm, table)
            for g in range(_NSG):
                pltpu.sync_copy(src_hbm.at[pl.ds(g * _SG, _SG)], sbuf)

                def body(kk, carry):
                    s = sbuf[pl.ds(kk * 16, 16)]
                    outb[pl.ds(kk * 16, 16)] = plsc.load_gather(table, [s])
                    return carry

                lax.fori_loop(0, _SG // 16, body, 0)
                pltpu.sync_copy(outb, tlu_hbm.at[pl.ds(g * _SG, _SG)])

        @pl.when(wid == 0)
        def _():
            for g in range(_NSG):
                pltpu.sync_copy(src_hbm.at[pl.ds(g * _SG, _SG)], sbuf)

                def p1(kk, carry):
                    s = sbuf[pl.ds(kk * 16, 16)]
                    plsc.store_scatter(table, [s],
                                       jnp.full((16,), -1, jnp.int32))
                    return carry

                lax.fori_loop(0, _SG // 16, p1, 0)

            for g in range(_NSG):
                pltpu.sync_copy(src_hbm.at[pl.ds(g * _SG, _SG)], sbuf)

                def p2(kk, carry):
                    s = sbuf[pl.ds(kk * 16, 16)]
                    pos = g * _SG + kk * 16 + lanes
                    key = s * B + pos
                    sk, sv = plsc.sort_key_val(key, s, descending=True)
                    prev = lax.gather(
                        sv, jnp.maximum(lanes - 1, 0)[:, None],
                        dimension_numbers=lax.GatherDimensionNumbers(
                            offset_dims=(), collapsed_slice_dims=(0,),
                            start_index_map=(0,)),
                        slice_sizes=(1,),
                        mode=lax.GatherScatterMode.PROMISE_IN_BOUNDS)
                    firstm = (lanes == 0) | (sv != prev)
                    cur = plsc.load_gather(table, [sv])
                    plsc.store_scatter(table, [sv], sk,
                                       mask=firstm & (sk > cur))
                    return carry

                lax.fori_loop(0, _SG // 16, p2, 0)

            for g in range(_NSG):
                pltpu.sync_copy(src_hbm.at[pl.ds(g * _SG, _SG)], sbuf)

                def p3(kk, carry):
                    s = sbuf[pl.ds(kk * 16, 16)]
                    pos = g * _SG + kk * 16 + lanes
                    key = s * B + pos
                    cur = plsc.load_gather(table, [s])
                    outb[pl.ds(kk * 16, 16)] = jnp.where(cur == key, s,
                                                         jnp.int32(-1))
                    return carry

                lax.fori_loop(0, _SG // 16, p3, 0)
                pltpu.sync_copy(outb, scat_hbm.at[pl.ds(g * _SG, _SG)])

    return k(sources, destinations, node_mem, lu_i32)


def _sc_gather_ef(edge_idxs, edge_features):
    per_tile = _NCHUNK // _NW

    @functools.partial(
        pl.kernel,
        mesh=_MESH,
        compiler_params=_CP,
        out_type=jax.ShapeDtypeStruct((B, MP), jnp.float32),
        scratch_types=[
            pltpu.VMEM((_RCH, MP), jnp.float32),
            pltpu.VMEM((_RCH,), jnp.int32),
            pltpu.SemaphoreType.DMA,
        ],
    )
    def k(eidx_hbm, ef_f, efg_hbm, rowbuf, idxv, sem):
        wid = lax.axis_index("s") * _NC + lax.axis_index("c")
        lanes = _lanes()
        for cch in range(per_tile):
            off = (wid * per_tile + cch) * _RCH
            pltpu.sync_copy(eidx_hbm.at[pl.ds(off, _RCH)], idxv)

            def fire(j, carry):
                g = (j // 16) * 16
                v = idxv[pl.ds(g, 16)]
                r = jnp.max(jnp.where(lanes == j - g, v, jnp.int32(-1)))
                pltpu.async_copy(ef_f.at[pl.ds(r, 1)],
                                 rowbuf.at[pl.ds(j, 1)], sem)
                return carry

            lax.fori_loop(0, _RCH, fire, 0)
            pltpu.make_async_copy(ef_f.at[pl.ds(0, _RCH)], rowbuf,
                                  sem).wait()
            pltpu.sync_copy(rowbuf, efg_hbm.at[pl.ds(off, _RCH)])

    return k(edge_idxs, edge_features)


_BLK = 1024
_GRID = B // _BLK


def _tc_dense(src_g, dst_g, ef_g, tlu, ts, w_src, w_dst, w_ef, w_t, bias,
              w_time_row, b_time_row):
    def body(src_ref, dst_ref, ef_ref, tlu_ref, ts_ref, wsrc_ref, wdst_ref,
             wef_ref, wt_ref, bias_ref, wtime_ref, btime_ref, out_ref):
        td = ts_ref[0, 0, :] - tlu_ref[0, 0, :]
        tenc = jnp.cos(td[:, None] * wtime_ref[0, :][None, :]
                       + btime_ref[0, :][None, :])
        h = jnp.dot(src_ref[...], wsrc_ref[...],
                    preferred_element_type=jnp.float32)
        h += jnp.dot(dst_ref[...], wdst_ref[...],
                     preferred_element_type=jnp.float32)
        h += jnp.dot(ef_ref[...], wef_ref[...],
                     preferred_element_type=jnp.float32)
        h += jnp.dot(tenc, wt_ref[...], preferred_element_type=jnp.float32)
        out_ref[...] = jnp.tanh(h + bias_ref[0, :][None, :])

    full = lambda shape: pl.BlockSpec(shape, lambda i: (0, 0))
    row = lambda shape: pl.BlockSpec(shape, lambda i: (i, 0))
    return pl.pallas_call(
        body,
        grid=(_GRID,),
        in_specs=[
            row((_BLK, MP)), row((_BLK, MP)), row((_BLK, MP)),
            pl.BlockSpec((1, 1, _BLK), lambda i: (i, 0, 0)),
            pl.BlockSpec((1, 1, _BLK), lambda i: (i, 0, 0)),
            full((MP, MP)), full((MP, MP)), full((MP, MP)),
            full((TENC, MP)), full((1, MP)), full((1, TENC)),
            full((1, TENC)),
        ],
        out_specs=row((_BLK, MP)),
        out_shape=jax.ShapeDtypeStruct((B, MP), jnp.float32),
    )(src_g, dst_g, ef_g, tlu.reshape(_GRID, 1, _BLK),
      ts.reshape(_GRID, 1, _BLK),
      w_src, w_dst, w_ef, w_t, bias, w_time_row, b_time_row)


_RANGE = N_NODES // _NW
_RBLK = 128
_NBLK = -(-_RANGE // _RBLK)
_WPAD = _NBLK * _RBLK


def _sc_scatter(h_new, scat_idx, node_mem):
    @functools.partial(
        pl.kernel,
        mesh=_MESH,
        compiler_params=_CP,
        out_type=jax.ShapeDtypeStruct((N_NODES, MP), jnp.float32),
        scratch_types=[
            pltpu.VMEM((_RBLK, MP), jnp.float32),
            pltpu.VMEM((_RBLK, MP), jnp.float32),
            pltpu.VMEM((_WPAD,), jnp.int32),
            pltpu.VMEM((_SG,), jnp.int32),
            pltpu.VMEM((1, MP), jnp.float32),
            pltpu.SemaphoreType.DMA,
            pltpu.SemaphoreType.DMA,
            pltpu.SemaphoreType.DMA,
        ],
    )
    def k(h_f, scat_hbm, mem_f, out_f, blockbuf_a, blockbuf_b, winmap,
          sbuf, dummy, sem, sem2a, sem2b):
        wid = lax.axis_index("s") * _NC + lax.axis_index("c")
        lanes = _lanes()
        base = wid * _RANGE

        def wm0(kk, carry):
            winmap[pl.ds(kk * 16, 16)] = jnp.full((16,), -1, jnp.int32)
            return carry

        lax.fori_loop(0, _WPAD // 16, wm0, 0)

        for g in range(_NSG):
            pltpu.sync_copy(scat_hbm.at[pl.ds(g * _SG, _SG)], sbuf)

            def scan(kk, carry):
                r = sbuf[pl.ds(kk * 16, 16)]
                pos = g * _SG + kk * 16 + lanes
                inr = (r >= base) & (r < base + _RANGE)
                plsc.store_scatter(winmap, [jnp.where(inr, r - base, 0)],
                                   pos, mask=inr)
                return carry

            lax.fori_loop(0, _SG // 16, scan, 0)

        bufs = (blockbuf_a, blockbuf_b)
        sems = (sem2a, sem2b)

        def bsize(b):
            return _RBLK if b < _NBLK - 1 else _RANGE - (_NBLK - 1) * _RBLK

        def load(b):
            pltpu.async_copy(
                mem_f.at[pl.ds(base + b * _RBLK, bsize(b))],
                bufs[b % 2].at[pl.ds(0, bsize(b))], sems[b % 2])

        load(0)
        for b in range(_NBLK):
            size = bsize(b)
            boff = b * _RBLK
            cur = bufs[b % 2]
            pltpu.make_async_copy(mem_f.at[pl.ds(0, size)],
                                  cur.at[pl.ds(0, size)],
                                  sems[b % 2]).wait()
            if b + 1 < _NBLK:
                load(b + 1)

            def fire(j, n):
                g = (j // 16) * 16
                v = winmap[pl.ds(boff + g, 16)]
                i_s = jnp.max(jnp.where(lanes == j - g, v, jnp.int32(-1)))

                @pl.when(i_s >= 0)
                def _():
                    pltpu.async_copy(h_f.at[pl.ds(i_s, 1)],
                                     cur.at[pl.ds(j, 1)], sem)

                return n + jnp.where(i_s >= 0, 1, 0)

            n = lax.fori_loop(0, size, fire, jnp.int32(0))

            def drain(j, carry):
                pltpu.make_async_copy(h_f.at[pl.ds(0, 1)], dummy,
                                      sem).wait()
                return carry

            lax.fori_loop(0, n, drain, 0)
            pltpu.sync_copy(cur.at[pl.ds(0, size)],
                            out_f.at[pl.ds(base + boff, size)])

    return k(h_new, scat_idx, node_mem)


def kernel(sources, destinations, timestamps, edge_idxs, edge_features,
           node_mem, last_updated, w_time, b_time, W_ih, W_hh, b_ih, b_hh):
    lu_i32 = lax.bitcast_convert_type(last_updated, jnp.int32)

    nm_row = _tc_transpose(node_mem.T)

    src_g, dst_g, tlu_i32, scat_idx = _sc_gather(
        sources, destinations, nm_row, lu_i32)

    ef_row = _tc_transpose(edge_features.T)
    ef_g = _sc_gather_ef(edge_idxs, ef_row)
    tlu = lax.bitcast_convert_type(tlu_i32, jnp.float32)

    pad = lambda w: jnp.pad(w, ((0, MP - w.shape[0]), (0, MP - w.shape[1])))
    w_src = pad((W_ih[:, :MEM] + W_hh).T)
    w_dst = pad(W_ih[:, MEM:2 * MEM].T)
    w_ef = pad(W_ih[:, 2 * MEM:2 * MEM + EF].T)
    w_t = jnp.pad(W_ih[:, 2 * MEM + EF:].T, ((0, 0), (0, MP - MEM)))
    bias = jnp.pad(b_ih + b_hh, (0, MP - MEM)).reshape(1, MP)
    w_time_row = w_time[:, 0].reshape(1, TENC)
    b_time_row = b_time.reshape(1, TENC)

    h_new = _tc_dense(src_g, dst_g, ef_g, tlu, timestamps, w_src, w_dst,
                      w_ef, w_t, bias, w_time_row, b_time_row)

    out = _sc_scatter(h_new, scat_idx, nm_row)
    return _tc_transpose_back(out).T


# speedup vs baseline: 1.2608x; 1.0623x over previous
import functools

import jax
import jax.numpy as jnp
from jax import lax
from jax.experimental import pallas as pl
from jax.experimental.pallas import tpu as pltpu, tpu_sc as plsc

B = 16384
MEM = 172
EF = 172
MP = 176
TENC = 100
N_NODES = 100000
N_EDGES = 400000

_NC = 2
_NS = 16
_NW = _NC * _NS

_RCH = 128
_NCHUNK = B // _RCH
_TOTAL = 2 * _NCHUNK
_GT = _NW - 2
_PER_TILE = -(-_TOTAL // _GT)

_SG = 512
_NSG = B // _SG

_CP = pltpu.CompilerParams(use_tc_tiling_on_sc=False,
                           needs_layout_passes=False)
_MESH = plsc.VectorSubcoreMesh(core_axis_name="c", subcore_axis_name="s")


def _lanes():
    return lax.iota(jnp.int32, 16)


_TBLK = 4096


def _tc_transpose(x_t):
    d, n = x_t.shape
    grid = -(-n // _TBLK)
    eye = jnp.eye(d, MP, dtype=x_t.dtype)

    def body(x_ref, eye_ref, o_ref):
        o_ref[...] = lax.dot_general(
            x_ref[...], eye_ref[...], (((0,), (0,)), ((), ())),
            preferred_element_type=jnp.float32)

    return pl.pallas_call(
        body,
        grid=(grid,),
        in_specs=[pl.BlockSpec((d, _TBLK), lambda i: (0, i)),
                  pl.BlockSpec((d, MP), lambda i: (0, 0))],
        out_specs=pl.BlockSpec((_TBLK, MP), lambda i: (i, 0)),
        out_shape=jax.ShapeDtypeStruct((n, MP), x_t.dtype),
    )(x_t, eye)


def _tc_transpose_back(x):
    n, d = x.shape
    grid = -(-n // _TBLK)
    eye = jnp.eye(MEM, MP, dtype=x.dtype)

    def body(x_ref, eye_ref, o_ref):
        o_ref[...] = lax.dot_general(
            eye_ref[...], x_ref[...], (((1,), (1,)), ((), ())),
            preferred_element_type=jnp.float32)

    return pl.pallas_call(
        body,
        grid=(grid,),
        in_specs=[pl.BlockSpec((_TBLK, MP), lambda i: (i, 0)),
                  pl.BlockSpec((MEM, MP), lambda i: (0, 0))],
        out_specs=pl.BlockSpec((MEM, _TBLK), lambda i: (0, i)),
        out_shape=jax.ShapeDtypeStruct((MEM, n), x.dtype),
    )(x, eye)


def _sc_gather(sources, destinations, node_mem, lu_i32):
    @functools.partial(
        pl.kernel,
        mesh=_MESH,
        compiler_params=_CP,
        out_type=(
            jax.ShapeDtypeStruct((B, MP), jnp.float32),
            jax.ShapeDtypeStruct((B, MP), jnp.float32),
            jax.ShapeDtypeStruct((B,), jnp.int32),
            jax.ShapeDtypeStruct((B,), jnp.int32),
        ),
        scratch_types=[
            pltpu.VMEM((N_NODES,), jnp.int32),
            pltpu.VMEM((_RCH, MP), jnp.float32),
            pltpu.VMEM((_RCH,), jnp.int32),
            pltpu.VMEM((_SG,), jnp.int32),
            pltpu.VMEM((_SG,), jnp.int32),
            pltpu.SemaphoreType.DMA,
        ],
    )
    def k(src_hbm, dst_hbm, mem_f, lu_hbm,
          srcg_hbm, dstg_hbm, tlu_hbm, scat_hbm,
          table, rowbuf, idxv, sbuf, outb, sem):
        wid = lax.axis_index("s") * _NC + lax.axis_index("c")
        lanes = _lanes()

        @pl.when(wid >= 2)
        def _():
            for cch in range(_PER_TILE):
                q = (wid - 2) * _PER_TILE + cch

                @pl.when(q < _TOTAL)
                def _():
                    typ = q // _NCHUNK
                    off = (q % _NCHUNK) * _RCH

                    @pl.when(typ == 0)
                    def _():
                        pltpu.sync_copy(src_hbm.at[pl.ds(off, _RCH)], idxv)

                    @pl.when(typ == 1)
                    def _():
                        pltpu.sync_copy(dst_hbm.at[pl.ds(off, _RCH)], idxv)

                    def fire(j, carry):
                        g = (j // 16) * 16
                        v = idxv[pl.ds(g, 16)]
                        r = jnp.max(jnp.where(lanes == j - g, v,
                                              jnp.int32(-1)))
                        pltpu.async_copy(mem_f.at[pl.ds(r, 1)],
                                         rowbuf.at[pl.ds(j, 1)], sem)
                        return carry

                    lax.fori_loop(0, _RCH, fire, 0)
                    pltpu.make_async_copy(mem_f.at[pl.ds(0, _RCH)],
                                          rowbuf, sem).wait()

                    @pl.when(typ == 0)
                    def _():
                        pltpu.sync_copy(rowbuf, srcg_hbm.at[pl.ds(off, _RCH)])

                    @pl.when(typ == 1)
                    def _():
                        pltpu.sync_copy(rowbuf, dstg_hbm.at[pl.ds(off, _RCH)])

        @pl.when(wid == 1)
        def _():
            pltpu.sync_copy(lu_hbm, table)
            for g in range(_NSG):
                pltpu.sync_copy(src_hbm.at[pl.ds(g * _SG, _SG)], sbuf)

                def body(kk, carry):
                    s = sbuf[pl.ds(kk * 16, 16)]
                    outb[pl.ds(kk * 16, 16)] = plsc.load_gather(table, [s])
                    return carry

                lax.fori_loop(0, _SG // 16, body, 0)
                pltpu.sync_copy(outb, tlu_hbm.at[pl.ds(g * _SG, _SG)])

        @pl.when(wid == 0)
        def _():
            for g in range(_NSG):
                pltpu.sync_copy(src_hbm.at[pl.ds(g * _SG, _SG)], sbuf)

                def p1(kk, carry):
                    s = sbuf[pl.ds(kk * 16, 16)]
                    plsc.store_scatter(table, [s],
                                       jnp.full((16,), -1, jnp.int32))
                    return carry

                lax.fori_loop(0, _SG // 16, p1, 0)

            for g in range(_NSG):
                pltpu.sync_copy(src_hbm.at[pl.ds(g * _SG, _SG)], sbuf)

                def p2(kk, carry):
                    s = sbuf[pl.ds(kk * 16, 16)]
                    pos = g * _SG + kk * 16 + lanes
                    key = s * B + pos
                    sk, sv = plsc.sort_key_val(key, s, descending=True)
                    prev = lax.gather(
                        sv, jnp.maximum(lanes - 1, 0)[:, None],
                        dimension_numbers=lax.GatherDimensionNumbers(
                            offset_dims=(), collapsed_slice_dims=(0,),
                            start_index_map=(0,)),
                        slice_sizes=(1,),
                        mode=lax.GatherScatterMode.PROMISE_IN_BOUNDS)
                    firstm = (lanes == 0) | (sv != prev)
                    cur = plsc.load_gather(table, [sv])
                    plsc.store_scatter(table, [sv], sk,
                                       mask=firstm & (sk > cur))
                    return carry

                lax.fori_loop(0, _SG // 16, p2, 0)

            for g in range(_NSG):
                pltpu.sync_copy(src_hbm.at[pl.ds(g * _SG, _SG)], sbuf)

                def p3(kk, carry):
                    s = sbuf[pl.ds(kk * 16, 16)]
                    pos = g * _SG + kk * 16 + lanes
                    key = s * B + pos
                    cur = plsc.load_gather(table, [s])
                    outb[pl.ds(kk * 16, 16)] = jnp.where(cur == key, s,
                                                         jnp.int32(-1))
                    return carry

                lax.fori_loop(0, _SG // 16, p3, 0)
                pltpu.sync_copy(outb, scat_hbm.at[pl.ds(g * _SG, _SG)])

    return k(sources, destinations, node_mem, lu_i32)


def _sc_gather_ef(edge_idxs, edge_features):
    per_tile = _NCHUNK // _NW

    @functools.partial(
        pl.kernel,
        mesh=_MESH,
        compiler_params=_CP,
        out_type=jax.ShapeDtypeStruct((B, MP), jnp.float32),
        scratch_types=[
            pltpu.VMEM((_RCH, MP), jnp.float32),
            pltpu.VMEM((_RCH,), jnp.int32),
            pltpu.SemaphoreType.DMA,
        ],
    )
    def k(eidx_hbm, ef_f, efg_hbm, rowbuf, idxv, sem):
        wid = lax.axis_index("s") * _NC + lax.axis_index("c")
        lanes = _lanes()
        for cch in range(per_tile):
            off = (wid * per_tile + cch) * _RCH
            pltpu.sync_copy(eidx_hbm.at[pl.ds(off, _RCH)], idxv)

            def fire(j, carry):
                g = (j // 16) * 16
                v = idxv[pl.ds(g, 16)]
                r = jnp.max(jnp.where(lanes == j - g, v, jnp.int32(-1)))
                pltpu.async_copy(ef_f.at[pl.ds(r, 1)],
                                 rowbuf.at[pl.ds(j, 1)], sem)
                return carry

            lax.fori_loop(0, _RCH, fire, 0)
            pltpu.make_async_copy(ef_f.at[pl.ds(0, _RCH)], rowbuf,
                                  sem).wait()
            pltpu.sync_copy(rowbuf, efg_hbm.at[pl.ds(off, _RCH)])

    return k(edge_idxs, edge_features)


_BLK = 1024
_GRID = B // _BLK


def _tc_dense(src_g, dst_g, ef_g, tlu, ts, w_src, w_dst, w_ef, w_t, bias,
              w_time_row, b_time_row):
    def body(src_ref, dst_ref, ef_ref, tlu_ref, ts_ref, wsrc_ref, wdst_ref,
             wef_ref, wt_ref, bias_ref, wtime_ref, btime_ref, out_ref):
        td = ts_ref[0, 0, :] - tlu_ref[0, 0, :]
        tenc = jnp.cos(td[:, None] * wtime_ref[0, :][None, :]
                       + btime_ref[0, :][None, :])
        h = jnp.dot(src_ref[...], wsrc_ref[...],
                    preferred_element_type=jnp.float32)
        h += jnp.dot(dst_ref[...], wdst_ref[...],
                     preferred_element_type=jnp.float32)
        h += jnp.dot(ef_ref[...], wef_ref[...],
                     preferred_element_type=jnp.float32)
        h += jnp.dot(tenc, wt_ref[...], preferred_element_type=jnp.float32)
        out_ref[...] = jnp.tanh(h + bias_ref[0, :][None, :])

    full = lambda shape: pl.BlockSpec(shape, lambda i: (0, 0))
    row = lambda shape: pl.BlockSpec(shape, lambda i: (i, 0))
    return pl.pallas_call(
        body,
        grid=(_GRID,),
        in_specs=[
            row((_BLK, MP)), row((_BLK, MP)), row((_BLK, MP)),
            pl.BlockSpec((1, 1, _BLK), lambda i: (i, 0, 0)),
            pl.BlockSpec((1, 1, _BLK), lambda i: (i, 0, 0)),
            full((MP, MP)), full((MP, MP)), full((MP, MP)),
            full((TENC, MP)), full((1, MP)), full((1, TENC)),
            full((1, TENC)),
        ],
        out_specs=row((_BLK, MP)),
        out_shape=jax.ShapeDtypeStruct((B, MP), jnp.float32),
    )(src_g, dst_g, ef_g, tlu.reshape(_GRID, 1, _BLK),
      ts.reshape(_GRID, 1, _BLK),
      w_src, w_dst, w_ef, w_t, bias, w_time_row, b_time_row)


_RANGE = N_NODES // _NW
_RBLK = 128
_NBLK = -(-_RANGE // _RBLK)
_WPAD = _NBLK * _RBLK


def _sc_scatter(h_new, scat_idx, node_mem):
    @functools.partial(
        pl.kernel,
        mesh=_MESH,
        compiler_params=_CP,
        out_type=jax.ShapeDtypeStruct((N_NODES, MP), jnp.float32),
        scratch_types=[
            pltpu.VMEM((_RBLK, MP), jnp.float32),
            pltpu.VMEM((_RBLK, MP), jnp.float32),
            pltpu.VMEM((_WPAD,), jnp.int32),
            pltpu.VMEM((_SG,), jnp.int32),
            pltpu.VMEM((1, MP), jnp.float32),
            pltpu.SemaphoreType.DMA,
            pltpu.SemaphoreType.DMA,
            pltpu.SemaphoreType.DMA,
        ],
    )
    def k(h_f, scat_hbm, mem_f, out_f, blockbuf_a, blockbuf_b, winmap,
          sbuf, dummy, sem, sem2a, sem2b):
        wid = lax.axis_index("s") * _NC + lax.axis_index("c")
        lanes = _lanes()
        base = wid * _RANGE

        def wm0(kk, carry):
            winmap[pl.ds(kk * 16, 16)] = jnp.full((16,), -1, jnp.int32)
            return carry

        lax.fori_loop(0, _WPAD // 16, wm0, 0)

        for g in range(_NSG):
            pltpu.sync_copy(scat_hbm.at[pl.ds(g * _SG, _SG)], sbuf)

            def scan(kk, carry):
                r = sbuf[pl.ds(kk * 16, 16)]
                pos = g * _SG + kk * 16 + lanes
                inr = (r >= base) & (r < base + _RANGE)
                plsc.store_scatter(winmap, [jnp.where(inr, r - base, 0)],
                                   pos, mask=inr)
                return carry

            lax.fori_loop(0, _SG // 16, scan, 0)

        bufs = (blockbuf_a, blockbuf_b)
        sems = (sem2a, sem2b)

        def bsize(b):
            return _RBLK if b < _NBLK - 1 else _RANGE - (_NBLK - 1) * _RBLK

        def load(b):
            pltpu.async_copy(
                mem_f.at[pl.ds(base + b * _RBLK, bsize(b))],
                bufs[b % 2].at[pl.ds(0, bsize(b))], sems[b % 2])

        load(0)
        for b in range(_NBLK):
            size = bsize(b)
            boff = b * _RBLK
            cur = bufs[b % 2]
            pltpu.make_async_copy(mem_f.at[pl.ds(0, size)],
                                  cur.at[pl.ds(0, size)],
                                  sems[b % 2]).wait()
            if b + 1 < _NBLK:
                load(b + 1)

            def fire(j, n):
                g = (j // 16) * 16
                v = winmap[pl.ds(boff + g, 16)]
                i_s = jnp.max(jnp.where(lanes == j - g, v, jnp.int32(-1)))

                @pl.when(i_s >= 0)
                def _():
                    pltpu.async_copy(h_f.at[pl.ds(i_s, 1)],
                                     cur.at[pl.ds(j, 1)], sem)

                return n + jnp.where(i_s >= 0, 1, 0)

            n = lax.fori_loop(0, size, fire, jnp.int32(0))

            def drain(j, carry):
                pltpu.make_async_copy(h_f.at[pl.ds(0, 1)], dummy,
                                      sem).wait()
                return carry

            lax.fori_loop(0, n, drain, 0)
            pltpu.sync_copy(cur.at[pl.ds(0, size)],
                            out_f.at[pl.ds(base + boff, size)])

    return k(h_new, scat_idx, node_mem)


def kernel(sources, destinations, timestamps, edge_idxs, edge_features,
           node_mem, last_updated, w_time, b_time, W_ih, W_hh, b_ih, b_hh):
    lu_i32 = lax.bitcast_convert_type(last_updated, jnp.int32)

    nm_row = _tc_transpose(node_mem.T)

    src_g, dst_g, tlu_i32, scat_idx = _sc_gather(
        sources, destinations, nm_row, lu_i32)

    ef_row = _tc_transpose(edge_features.T)
    ef_g = _sc_gather_ef(edge_idxs, ef_row)
    tlu = lax.bitcast_convert_type(tlu_i32, jnp.float32)

    pad = lambda w: jnp.pad(w, ((0, MP - w.shape[0]), (0, MP - w.shape[1])))
    w_src = pad((W_ih[:, :MEM] + W_hh).T)
    w_dst = pad(W_ih[:, MEM:2 * MEM].T)
    w_ef = pad(W_ih[:, 2 * MEM:2 * MEM + EF].T)
    w_t = jnp.pad(W_ih[:, 2 * MEM + EF:].T, ((0, 0), (0, MP - MEM)))
    bias = jnp.pad(b_ih + b_hh, (0, MP - MEM)).reshape(1, MP)
    w_time_row = w_time[:, 0].reshape(1, TENC)
    b_time_row = b_time.reshape(1, TENC)

    h_new = _tc_dense(src_g, dst_g, ef_g, tlu, timestamps, w_src, w_dst,
                      w_ef, w_t, bias, w_time_row, b_time_row)

    out = _sc_scatter(h_new, scat_idx, nm_row)
    return _tc_transpose_back(out).T


# speedup vs baseline: 1.2781x; 1.0138x over previous
import functools

import jax
import jax.numpy as jnp
from jax import lax
from jax.experimental import pallas as pl
from jax.experimental.pallas import tpu as pltpu, tpu_sc as plsc

B = 16384
MEM = 172
EF = 172
MP = 176
TENC = 100
N_NODES = 100000
N_EDGES = 400000

_NC = 2
_NS = 16
_NW = _NC * _NS

_RCH = 128
_NCHUNK = B // _RCH
_TOTAL = 2 * _NCHUNK
_GT = _NW - 2
_PER_TILE = -(-_TOTAL // _GT)

_SG = 512
_NSG = B // _SG

_CP = pltpu.CompilerParams(use_tc_tiling_on_sc=False,
                           needs_layout_passes=False)
_MESH = plsc.VectorSubcoreMesh(core_axis_name="c", subcore_axis_name="s")


def _lanes():
    return lax.iota(jnp.int32, 16)


_TBLK = 8192


def _tc_transpose(x_t):
    d, n = x_t.shape
    grid = -(-n // _TBLK)
    eye = jnp.eye(d, MP, dtype=x_t.dtype)

    def body(x_ref, eye_ref, o_ref):
        o_ref[...] = lax.dot_general(
            x_ref[...], eye_ref[...], (((0,), (0,)), ((), ())),
            preferred_element_type=jnp.float32)

    return pl.pallas_call(
        body,
        grid=(grid,),
        in_specs=[pl.BlockSpec((d, _TBLK), lambda i: (0, i)),
                  pl.BlockSpec((d, MP), lambda i: (0, 0))],
        out_specs=pl.BlockSpec((_TBLK, MP), lambda i: (i, 0)),
        out_shape=jax.ShapeDtypeStruct((n, MP), x_t.dtype),
    )(x_t, eye)


def _tc_transpose_back(x):
    n, d = x.shape
    grid = -(-n // _TBLK)
    eye = jnp.eye(MEM, MP, dtype=x.dtype)

    def body(x_ref, eye_ref, o_ref):
        o_ref[...] = lax.dot_general(
            eye_ref[...], x_ref[...], (((1,), (1,)), ((), ())),
            preferred_element_type=jnp.float32)

    return pl.pallas_call(
        body,
        grid=(grid,),
        in_specs=[pl.BlockSpec((_TBLK, MP), lambda i: (i, 0)),
                  pl.BlockSpec((MEM, MP), lambda i: (0, 0))],
        out_specs=pl.BlockSpec((MEM, _TBLK), lambda i: (0, i)),
        out_shape=jax.ShapeDtypeStruct((MEM, n), x.dtype),
    )(x, eye)


def _sc_gather(sources, destinations, node_mem, lu_i32):
    @functools.partial(
        pl.kernel,
        mesh=_MESH,
        compiler_params=_CP,
        out_type=(
            jax.ShapeDtypeStruct((B, MP), jnp.float32),
            jax.ShapeDtypeStruct((B, MP), jnp.float32),
            jax.ShapeDtypeStruct((B,), jnp.int32),
            jax.ShapeDtypeStruct((B,), jnp.int32),
        ),
        scratch_types=[
            pltpu.VMEM((N_NODES,), jnp.int32),
            pltpu.VMEM((_RCH, MP), jnp.float32),
            pltpu.VMEM((_RCH,), jnp.int32),
            pltpu.VMEM((_SG,), jnp.int32),
            pltpu.VMEM((_SG,), jnp.int32),
            pltpu.SemaphoreType.DMA,
        ],
    )
    def k(src_hbm, dst_hbm, mem_f, lu_hbm,
          srcg_hbm, dstg_hbm, tlu_hbm, scat_hbm,
          table, rowbuf, idxv, sbuf, outb, sem):
        wid = lax.axis_index("s") * _NC + lax.axis_index("c")
        lanes = _lanes()

        @pl.when(wid >= 2)
        def _():
            for cch in range(_PER_TILE):
                q = (wid - 2) * _PER_TILE + cch

                @pl.when(q < _TOTAL)
                def _():
                    typ = q // _NCHUNK
                    off = (q % _NCHUNK) * _RCH

                    @pl.when(typ == 0)
                    def _():
                        pltpu.sync_copy(src_hbm.at[pl.ds(off, _RCH)], idxv)

                    @pl.when(typ == 1)
                    def _():
                        pltpu.sync_copy(dst_hbm.at[pl.ds(off, _RCH)], idxv)

                    def fire(j, carry):
                        g = (j // 16) * 16
                        v = idxv[pl.ds(g, 16)]
                        r = jnp.max(jnp.where(lanes == j - g, v,
                                              jnp.int32(-1)))
                        pltpu.async_copy(mem_f.at[pl.ds(r, 1)],
                                         rowbuf.at[pl.ds(j, 1)], sem)
                        return carry

                    lax.fori_loop(0, _RCH, fire, 0)
                    pltpu.make_async_copy(mem_f.at[pl.ds(0, _RCH)],
                                          rowbuf, sem).wait()

                    @pl.when(typ == 0)
                    def _():
                        pltpu.sync_copy(rowbuf, srcg_hbm.at[pl.ds(off, _RCH)])

                    @pl.when(typ == 1)
                    def _():
                        pltpu.sync_copy(rowbuf, dstg_hbm.at[pl.ds(off, _RCH)])

        @pl.when(wid == 1)
        def _():
            pltpu.sync_copy(lu_hbm, table)
            for g in range(_NSG):
                pltpu.sync_copy(src_hbm.at[pl.ds(g * _SG, _SG)], sbuf)

                def body(kk, carry):
                    s = sbuf[pl.ds(kk * 16, 16)]
                    outb[pl.ds(kk * 16, 16)] = plsc.load_gather(table, [s])
                    return carry

                lax.fori_loop(0, _SG // 16, body, 0)
                pltpu.sync_copy(outb, tlu_hbm.at[pl.ds(g * _SG, _SG)])

        @pl.when(wid == 0)
        def _():
            for g in range(_NSG):
                pltpu.sync_copy(src_hbm.at[pl.ds(g * _SG, _SG)], sbuf)

                def p1(kk, carry):
                    s = sbuf[pl.ds(kk * 16, 16)]
                    plsc.store_scatter(table, [s],
                                       jnp.full((16,), -1, jnp.int32))
                    return carry

                lax.fori_loop(0, _SG // 16, p1, 0)

            for g in range(_NSG):
                pltpu.sync_copy(src_hbm.at[pl.ds(g * _SG, _SG)], sbuf)

                def p2(kk, carry):
                    s = sbuf[pl.ds(kk * 16, 16)]
                    pos = g * _SG + kk * 16 + lanes
                    key = s * B + pos
                    sk, sv = plsc.sort_key_val(key, s, descending=True)
                    prev = lax.gather(
                        sv, jnp.maximum(lanes - 1, 0)[:, None],
                        dimension_numbers=lax.GatherDimensionNumbers(
                            offset_dims=(), collapsed_slice_dims=(0,),
                            start_index_map=(0,)),
                        slice_sizes=(1,),
                        mode=lax.GatherScatterMode.PROMISE_IN_BOUNDS)
                    firstm = (lanes == 0) | (sv != prev)
                    cur = plsc.load_gather(table, [sv])
                    plsc.store_scatter(table, [sv], sk,
                                       mask=firstm & (sk > cur))
                    return carry

                lax.fori_loop(0, _SG // 16, p2, 0)

            for g in range(_NSG):
                pltpu.sync_copy(src_hbm.at[pl.ds(g * _SG, _SG)], sbuf)

                def p3(kk, carry):
                    s = sbuf[pl.ds(kk * 16, 16)]
                    pos = g * _SG + kk * 16 + lanes
                    key = s * B + pos
                    cur = plsc.load_gather(table, [s])
                    outb[pl.ds(kk * 16, 16)] = jnp.where(cur == key, s,
                                                         jnp.int32(-1))
                    return carry

                lax.fori_loop(0, _SG // 16, p3, 0)
                pltpu.sync_copy(outb, scat_hbm.at[pl.ds(g * _SG, _SG)])

    return k(sources, destinations, node_mem, lu_i32)


def _sc_gather_ef(edge_idxs, edge_features):
    per_tile = _NCHUNK // _NW

    @functools.partial(
        pl.kernel,
        mesh=_MESH,
        compiler_params=_CP,
        out_type=jax.ShapeDtypeStruct((B, MP), jnp.float32),
        scratch_types=[
            pltpu.VMEM((_RCH, MP), jnp.float32),
            pltpu.VMEM((_RCH,), jnp.int32),
            pltpu.SemaphoreType.DMA,
        ],
    )
    def k(eidx_hbm, ef_f, efg_hbm, rowbuf, idxv, sem):
        wid = lax.axis_index("s") * _NC + lax.axis_index("c")
        lanes = _lanes()
        for cch in range(per_tile):
            off = (wid * per_tile + cch) * _RCH
            pltpu.sync_copy(eidx_hbm.at[pl.ds(off, _RCH)], idxv)

            def fire(j, carry):
                g = (j // 16) * 16
                v = idxv[pl.ds(g, 16)]
                r = jnp.max(jnp.where(lanes == j - g, v, jnp.int32(-1)))
                pltpu.async_copy(ef_f.at[pl.ds(r, 1)],
                                 rowbuf.at[pl.ds(j, 1)], sem)
                return carry

            lax.fori_loop(0, _RCH, fire, 0)
            pltpu.make_async_copy(ef_f.at[pl.ds(0, _RCH)], rowbuf,
                                  sem).wait()
            pltpu.sync_copy(rowbuf, efg_hbm.at[pl.ds(off, _RCH)])

    return k(edge_idxs, edge_features)


_BLK = 1024
_GRID = B // _BLK


def _tc_dense(src_g, dst_g, ef_g, tlu, ts, w_src, w_dst, w_ef, w_t, bias,
              w_time_row, b_time_row):
    def body(src_ref, dst_ref, ef_ref, tlu_ref, ts_ref, wsrc_ref, wdst_ref,
             wef_ref, wt_ref, bias_ref, wtime_ref, btime_ref, out_ref):
        td = ts_ref[0, 0, :] - tlu_ref[0, 0, :]
        tenc = jnp.cos(td[:, None] * wtime_ref[0, :][None, :]
                       + btime_ref[0, :][None, :])
        h = jnp.dot(src_ref[...], wsrc_ref[...],
                    preferred_element_type=jnp.float32)
        h += jnp.dot(dst_ref[...], wdst_ref[...],
                     preferred_element_type=jnp.float32)
        h += jnp.dot(ef_ref[...], wef_ref[...],
                     preferred_element_type=jnp.float32)
        h += jnp.dot(tenc, wt_ref[...], preferred_element_type=jnp.float32)
        out_ref[...] = jnp.tanh(h + bias_ref[0, :][None, :])

    full = lambda shape: pl.BlockSpec(shape, lambda i: (0, 0))
    row = lambda shape: pl.BlockSpec(shape, lambda i: (i, 0))
    return pl.pallas_call(
        body,
        grid=(_GRID,),
        in_specs=[
            row((_BLK, MP)), row((_BLK, MP)), row((_BLK, MP)),
            pl.BlockSpec((1, 1, _BLK), lambda i: (i, 0, 0)),
            pl.BlockSpec((1, 1, _BLK), lambda i: (i, 0, 0)),
            full((MP, MP)), full((MP, MP)), full((MP, MP)),
            full((TENC, MP)), full((1, MP)), full((1, TENC)),
            full((1, TENC)),
        ],
        out_specs=row((_BLK, MP)),
        out_shape=jax.ShapeDtypeStruct((B, MP), jnp.float32),
    )(src_g, dst_g, ef_g, tlu.reshape(_GRID, 1, _BLK),
      ts.reshape(_GRID, 1, _BLK),
      w_src, w_dst, w_ef, w_t, bias, w_time_row, b_time_row)


_RANGE = N_NODES // _NW
_RBLK = 128
_NBLK = -(-_RANGE // _RBLK)
_WPAD = _NBLK * _RBLK


def _sc_scatter(h_new, scat_idx, node_mem):
    @functools.partial(
        pl.kernel,
        mesh=_MESH,
        compiler_params=_CP,
        out_type=jax.ShapeDtypeStruct((N_NODES, MP), jnp.float32),
        scratch_types=[
            pltpu.VMEM((_RBLK, MP), jnp.float32),
            pltpu.VMEM((_RBLK, MP), jnp.float32),
            pltpu.VMEM((_WPAD,), jnp.int32),
            pltpu.VMEM((_SG,), jnp.int32),
            pltpu.VMEM((1, MP), jnp.float32),
            pltpu.SemaphoreType.DMA,
            pltpu.SemaphoreType.DMA,
            pltpu.SemaphoreType.DMA,
        ],
    )
    def k(h_f, scat_hbm, mem_f, out_f, blockbuf_a, blockbuf_b, winmap,
          sbuf, dummy, sem, sem2a, sem2b):
        wid = lax.axis_index("s") * _NC + lax.axis_index("c")
        lanes = _lanes()
        base = wid * _RANGE

        def wm0(kk, carry):
            winmap[pl.ds(kk * 16, 16)] = jnp.full((16,), -1, jnp.int32)
            return carry

        lax.fori_loop(0, _WPAD // 16, wm0, 0)

        for g in range(_NSG):
            pltpu.sync_copy(scat_hbm.at[pl.ds(g * _SG, _SG)], sbuf)

            def scan(kk, carry):
                r = sbuf[pl.ds(kk * 16, 16)]
                pos = g * _SG + kk * 16 + lanes
                inr = (r >= base) & (r < base + _RANGE)
                plsc.store_scatter(winmap, [jnp.where(inr, r - base, 0)],
                                   pos, mask=inr)
                return carry

            lax.fori_loop(0, _SG // 16, scan, 0)

        bufs = (blockbuf_a, blockbuf_b)
        sems = (sem2a, sem2b)

        def bsize(b):
            return _RBLK if b < _NBLK - 1 else _RANGE - (_NBLK - 1) * _RBLK

        def load(b):
            pltpu.async_copy(
                mem_f.at[pl.ds(base + b * _RBLK, bsize(b))],
                bufs[b % 2].at[pl.ds(0, bsize(b))], sems[b % 2])

        load(0)
        for b in range(_NBLK):
            size = bsize(b)
            boff = b * _RBLK
            cur = bufs[b % 2]
            pltpu.make_async_copy(mem_f.at[pl.ds(0, size)],
                                  cur.at[pl.ds(0, size)],
                                  sems[b % 2]).wait()
            if b + 1 < _NBLK:
                load(b + 1)

            def fire(j, n):
                g = (j // 16) * 16
                v = winmap[pl.ds(boff + g, 16)]
                i_s = jnp.max(jnp.where(lanes == j - g, v, jnp.int32(-1)))

                @pl.when(i_s >= 0)
                def _():
                    pltpu.async_copy(h_f.at[pl.ds(i_s, 1)],
                                     cur.at[pl.ds(j, 1)], sem)

                return n + jnp.where(i_s >= 0, 1, 0)

            n = lax.fori_loop(0, size, fire, jnp.int32(0))

            def drain(j, carry):
                pltpu.make_async_copy(h_f.at[pl.ds(0, 1)], dummy,
                                      sem).wait()
                return carry

            lax.fori_loop(0, n, drain, 0)
            pltpu.sync_copy(cur.at[pl.ds(0, size)],
                            out_f.at[pl.ds(base + boff, size)])

    return k(h_new, scat_idx, node_mem)


def kernel(sources, destinations, timestamps, edge_idxs, edge_features,
           node_mem, last_updated, w_time, b_time, W_ih, W_hh, b_ih, b_hh):
    lu_i32 = lax.bitcast_convert_type(last_updated, jnp.int32)

    nm_row = _tc_transpose(node_mem.T)

    src_g, dst_g, tlu_i32, scat_idx = _sc_gather(
        sources, destinations, nm_row, lu_i32)

    ef_row = _tc_transpose(edge_features.T)
    ef_g = _sc_gather_ef(edge_idxs, ef_row)
    tlu = lax.bitcast_convert_type(tlu_i32, jnp.float32)

    pad = lambda w: jnp.pad(w, ((0, MP - w.shape[0]), (0, MP - w.shape[1])))
    w_src = pad((W_ih[:, :MEM] + W_hh).T)
    w_dst = pad(W_ih[:, MEM:2 * MEM].T)
    w_ef = pad(W_ih[:, 2 * MEM:2 * MEM + EF].T)
    w_t = jnp.pad(W_ih[:, 2 * MEM + EF:].T, ((0, 0), (0, MP - MEM)))
    bias = jnp.pad(b_ih + b_hh, (0, MP - MEM)).reshape(1, MP)
    w_time_row = w_time[:, 0].reshape(1, TENC)
    b_time_row = b_time.reshape(1, TENC)

    h_new = _tc_dense(src_g, dst_g, ef_g, tlu, timestamps, w_src, w_dst,
                      w_ef, w_t, bias, w_time_row, b_time_row)

    out = _sc_scatter(h_new, scat_idx, nm_row)
    return _tc_transpose_back(out).T
